# baseline (speedup 1.0000x reference)
import functools
import math

import jax
import jax.numpy as jnp
from jax import lax
from jax.experimental import pallas as pl
from jax.experimental.pallas import tpu as pltpu
from jax.experimental.pallas import tpu_sc as plsc


@functools.lru_cache(maxsize=None)
def _make_gather(V, D, B):
    info = plsc.get_sparse_core_info()
    NC, NS, L = info.num_cores, info.num_subcores, info.num_lanes
    NW = NC * NS
    assert D % L == 0 and B % (8 * NW) == 0
    b_per_w = B // NW
    C = 16
    n_chunks = b_per_w // C
    n_groups = n_chunks // 2
    scale = math.sqrt(D)
    mesh = plsc.VectorSubcoreMesh(core_axis_name="c", subcore_axis_name="s")

    NA = 4
    NB = 2

    @functools.partial(
        pl.kernel,
        mesh=mesh,
        out_type=jax.ShapeDtypeStruct((B, D), jnp.float32),
        scratch_types=[
            pltpu.VMEM((b_per_w,), jnp.int32),
        ]
        + [pltpu.VMEM((C, D), jnp.float32) for _ in range(NA + NB)]
        + [pltpu.SemaphoreType.DMA for _ in range(NA + NB)],
    )
    def gather_scale(table_hbm, idx_hbm, out_hbm, idx_v, *rest):
        abuf = rest[:NA]
        bbuf = rest[NA : NA + NB]
        gsem = rest[NA + NB : 2 * NA + NB]
        ssem = rest[2 * NA + NB : 2 * NA + 2 * NB]
        wid = lax.axis_index("s") * NC + lax.axis_index("c")
        base = wid * b_per_w
        pltpu.sync_copy(idx_hbm.at[pl.ds(base, b_per_w)], idx_v)

        def gather_descr(c, a):
            return pltpu.make_async_copy(
                table_hbm.at[idx_v.at[pl.ds(c * C, C)]], abuf[a], gsem[a]
            )

        def store_descr(c, b):
            return pltpu.make_async_copy(
                bbuf[b], out_hbm.at[pl.ds(base + c * C, C)], ssem[b]
            )

        def phase(c, a, b):
            gather_descr(c, a).wait()

            @pl.when(c >= NB)
            def _():
                store_descr(c, b).wait()

            @plsc.parallel_loop(0, C, 1)
            def _(i):
                for j in range(D // L):
                    bbuf[b][i, pl.ds(j * L, L)] = (
                        abuf[a][i, pl.ds(j * L, L)] * scale
                    )

            store_descr(c, b).start()

            @pl.when(c + NA < n_chunks)
            def _():
                gather_descr(c + NA, a).start()

        for a in range(NA):
            gather_descr(a, a).start()

        def group_body(g, carry):
            for k in range(NA):
                c = g * NA + k
                phase(c, k, k % NB)
            return carry

        lax.fori_loop(0, n_chunks // NA, group_body, 0)
        for b in range(NB):
            store_descr(0, b).wait()

    return gather_scale


def kernel(x, emb):
    V, D = emb.shape
    B = x.size
    x_flat = x.reshape(B).astype(jnp.int32)
    out = _make_gather(V, D, B)(emb, x_flat)
    return out.reshape(x.shape + (D,))
